# baseline (speedup 1.0000x reference)
import jax
import jax.numpy as jnp
from jax import lax
from jax.experimental import pallas as pl
from jax.experimental.pallas import tpu as pltpu, tpu_sc as plsc

_LANES = 16
_NC, _NS = 2, 16
_NW = _NC * _NS
_NSPLIT = 2


def _pop_gather_body(freq_hbm, cand_hbm, out_hbm, table_sh, idx0, idx1,
                     rows0, rows1, gsem, ssem):
    sid = lax.axis_index("s")
    wid = sid * _NC + lax.axis_index("c")
    idx = (idx0, idx1)
    rows = (rows0, rows1)

    @pl.when(sid == 0)
    def _stage():
        pltpu.sync_copy(freq_hbm, table_sh)

    for j in range(_NSPLIT):
        pltpu.sync_copy(cand_hbm.at[wid * _NSPLIT + j], idx[j])
    plsc.subcore_barrier()

    gathers = []
    for j in range(_NSPLIT):
        gathers.append(pltpu.async_copy(table_sh.at[idx[j]], rows[j], gsem))
    scatters = []
    for j in range(_NSPLIT):
        gathers[j].wait()
        scatters.append(pltpu.async_copy(
            rows[j], out_hbm.at[wid * _NSPLIT + j], ssem))
    for s in scatters:
        s.wait()


def kernel(tokens, candidates, item_freq):
    del tokens
    b, ncand = candidates.shape
    total = b * ncand
    vocab = item_freq.shape[-1]
    chunk = total // _NW
    sub = chunk // _NSPLIT
    assert total % (_NW * _NSPLIT) == 0 and sub % _LANES == 0
    assert (sub * 4) % 64 == 0

    mesh = plsc.VectorSubcoreMesh(
        core_axis_name="c", subcore_axis_name="s",
        num_cores=_NC, num_subcores=_NS)
    run = pl.kernel(
        _pop_gather_body,
        out_type=jax.ShapeDtypeStruct((_NW * _NSPLIT, sub), jnp.float32),
        mesh=mesh,
        scratch_types=[
            pltpu.VMEM_SHARED((vocab,), jnp.float32),
            pltpu.VMEM((sub,), jnp.int32),
            pltpu.VMEM((sub,), jnp.int32),
            pltpu.VMEM((sub,), jnp.float32),
            pltpu.VMEM((sub,), jnp.float32),
            pltpu.SemaphoreType.DMA,
            pltpu.SemaphoreType.DMA,
        ],
        compiler_params=pltpu.CompilerParams(needs_layout_passes=False),
    )
    out = run(item_freq.reshape(vocab),
              candidates.reshape(_NW * _NSPLIT, sub))
    out = out.reshape(b, ncand)
    return (out, out)


# speedup vs baseline: 1.0056x; 1.0056x over previous
import jax
import jax.numpy as jnp
from jax import lax
from jax.experimental import pallas as pl
from jax.experimental.pallas import tpu as pltpu, tpu_sc as plsc

_LANES = 16
_NC, _NS = 2, 16
_NW = _NC * _NS


def _pop_gather_body(freq_hbm, cand_hbm, out_hbm, table_sh, idx_v, rows_v, sem):
    sid = lax.axis_index("s")
    wid = sid * _NC + lax.axis_index("c")
    chunk = idx_v.shape[0]
    base = wid * chunk

    @pl.when(sid == 0)
    def _stage():
        pltpu.sync_copy(freq_hbm, table_sh)

    pltpu.sync_copy(cand_hbm.at[pl.ds(base, chunk)], idx_v)
    plsc.subcore_barrier()
    pltpu.async_copy(table_sh.at[idx_v], rows_v, sem).wait()
    pltpu.sync_copy(rows_v, out_hbm.at[pl.ds(base, chunk)])


def kernel(tokens, candidates, item_freq):
    del tokens
    b, ncand = candidates.shape
    total = b * ncand
    vocab = item_freq.shape[-1]
    chunk = total // _NW
    assert total % (_NW * _LANES) == 0 and chunk % 8 == 0

    mesh = plsc.VectorSubcoreMesh(
        core_axis_name="c", subcore_axis_name="s",
        num_cores=_NC, num_subcores=_NS)
    run = pl.kernel(
        _pop_gather_body,
        out_type=jax.ShapeDtypeStruct((total,), jnp.float32),
        mesh=mesh,
        scratch_types=[
            pltpu.VMEM_SHARED((vocab,), jnp.float32),
            pltpu.VMEM((chunk,), jnp.int32),
            pltpu.VMEM((chunk,), jnp.float32),
            pltpu.SemaphoreType.DMA,
        ],
        compiler_params=pltpu.CompilerParams(needs_layout_passes=False),
    )
    out = run(item_freq.reshape(vocab), candidates.reshape(total))
    out = out.reshape(b, ncand)
    return (out, out)


# speedup vs baseline: 1.0079x; 1.0024x over previous
import jax
import jax.numpy as jnp
from jax import lax
from jax.experimental import pallas as pl
from jax.experimental.pallas import tpu as pltpu, tpu_sc as plsc

_LANES = 16
_NC, _NS = 2, 16
_NW = _NC * _NS


def _pop_gather_body(freq_hbm, cand_hbm, out_hbm, table_sh, idx_v, rows_v, sem):
    sid = lax.axis_index("s")
    wid = sid * _NC + lax.axis_index("c")
    chunk = idx_v.shape[0]
    base = wid * chunk

    @pl.when(sid == 0)
    def _stage():
        pltpu.sync_copy(freq_hbm, table_sh)

    pltpu.sync_copy(cand_hbm.at[pl.ds(base, chunk)], idx_v)
    plsc.subcore_barrier()
    pltpu.async_copy(table_sh.at[idx_v], rows_v, sem).wait()
    pltpu.sync_copy(rows_v, out_hbm.at[pl.ds(base, chunk)])


def kernel(tokens, candidates, item_freq):
    del tokens
    b, ncand = candidates.shape
    total = b * ncand
    vocab = item_freq.shape[-1]
    chunk = total // _NW
    assert total % (_NW * _LANES) == 0 and chunk % 8 == 0

    mesh = plsc.VectorSubcoreMesh(
        core_axis_name="c", subcore_axis_name="s",
        num_cores=_NC, num_subcores=_NS)
    run = pl.kernel(
        _pop_gather_body,
        out_type=jax.ShapeDtypeStruct((total,), jnp.float32),
        mesh=mesh,
        scratch_types=[
            pltpu.VMEM_SHARED((vocab,), jnp.float32),
            pltpu.VMEM((chunk,), jnp.int32),
            pltpu.VMEM((chunk,), jnp.float32),
            pltpu.SemaphoreType.DMA,
        ],
        compiler_params=pltpu.CompilerParams(needs_layout_passes=False),
    )
    out = run(item_freq[0], candidates.reshape(total))
    out = out.reshape(b, ncand)
    return (out, out)
